# baseline (speedup 1.0000x reference)
import functools

import jax
import jax.numpy as jnp
from jax import lax
from jax.experimental import pallas as pl
from jax.experimental.pallas import tpu as pltpu
from jax.experimental.pallas import tpu_sc as plsc

_B, _S, _D = 4, 4096, 1024
_N = _B * _S
_NC, _NS = 2, 16
_NW = _NC * _NS
_CHUNKS = (2048, 4096, 4096, 4096, 2048)
_GC = 16
_LANES = 16
_ROWS = 1024
_EPS = 1e-12


def _sc_gather_chunk(cbase, ntok):
    mesh = plsc.VectorSubcoreMesh(core_axis_name="c", subcore_axis_name="s")
    _PER_W = ntok // _NW
    _NG = _PER_W // _GC

    @functools.partial(
        pl.kernel,
        mesh=mesh,
        out_type=jax.ShapeDtypeStruct((ntok, _D), jnp.float32),
        scratch_types=[
            pltpu.VMEM((_PER_W,), jnp.int32),
            pltpu.VMEM((_PER_W,), jnp.int32),
        ]
        + [pltpu.VMEM((_GC, _D), jnp.float32)] * 4
        + [pltpu.SemaphoreType.DMA] * 6,
    )
    def k(pos_tab, atom_tab, pid, aid, sum_out, pidx_v, aidx_v, *scr):
        bufp = scr[0:2]
        bufa = scr[2:4]
        gsemp = scr[4:6]
        gsema = scr[6:8]
        wsem = scr[8:10]
        wid = lax.axis_index("s") * _NC + lax.axis_index("c")
        base = wid * _PER_W
        pltpu.sync_copy(pid.at[pl.ds(cbase + base, _PER_W)], pidx_v)
        pltpu.sync_copy(aid.at[pl.ds(cbase + base, _PER_W)], aidx_v)

        def _gcopies(c):
            b = c % 2
            off = c * _GC
            return (
                pltpu.make_async_copy(
                    pos_tab.at[pidx_v.at[pl.ds(off, _GC)]], bufp[b], gsemp[b]
                ),
                pltpu.make_async_copy(
                    atom_tab.at[aidx_v.at[pl.ds(off, _GC)]], bufa[b], gsema[b]
                ),
            )

        def _wcopy(c):
            b = c % 2
            off = c * _GC
            return pltpu.make_async_copy(
                bufp[b], sum_out.at[pl.ds(base + off, _GC)], wsem[b]
            )

        def _add(c):
            b = c % 2
            bp, ba = bufp[b], bufa[b]

            @pl.loop(0, _GC)
            def _(r):
                for u in range(_D // _LANES):
                    sl = pl.ds(u * _LANES, _LANES)
                    bp[r, sl] = bp[r, sl] + ba[r, sl]

        def _process(c):
            for cp in _gcopies(c):
                cp.wait()
            _add(c)
            _wcopy(c).start()

        for c in range(_NG):
            if c >= 2:
                _wcopy(c - 2).wait()
            for cp in _gcopies(c):
                cp.start()
            if c >= 1:
                _process(c - 1)
        _process(_NG - 1)
        _wcopy(_NG - 2).wait()
        _wcopy(_NG - 1).wait()

    return k


def _addln_body(x_ref, s_ref, g_ref, b_ref, o_ref):
    v = x_ref[...] + s_ref[...]
    mean = jnp.mean(v, axis=-1, keepdims=True)
    vc = v - mean
    var = jnp.mean(vc * vc, axis=-1, keepdims=True)
    o_ref[...] = vc * lax.rsqrt(var + _EPS) * g_ref[...] + b_ref[...]


def _tc_chunk(rbase, ntok, buf, x, sum_j, gamma, beta):
    blk0 = rbase // _ROWS
    row_spec = pl.BlockSpec((_ROWS, _D), lambda i, blk0=blk0: (blk0 + i, 0))
    chunk_spec = pl.BlockSpec((_ROWS, _D), lambda i: (i, 0))
    vec_spec = pl.BlockSpec((1, _D), lambda i: (0, 0))
    common = dict(
        grid=(ntok // _ROWS,),
        out_specs=row_spec,
        out_shape=jax.ShapeDtypeStruct((_N, _D), jnp.float32),
    )
    if buf is None:
        return pl.pallas_call(
            _addln_body,
            in_specs=[row_spec, chunk_spec, vec_spec, vec_spec],
            **common,
        )(x, sum_j, gamma, beta)

    def body(buf_ref, x_ref, s_ref, g_ref, b_ref, o_ref):
        _addln_body(x_ref, s_ref, g_ref, b_ref, o_ref)

    return pl.pallas_call(
        body,
        in_specs=[
            pl.BlockSpec(memory_space=pl.ANY),
            row_spec,
            chunk_spec,
            vec_spec,
            vec_spec,
        ],
        input_output_aliases={0: 0},
        **common,
    )(buf, x, sum_j, gamma, beta)


def kernel(input_embeds, position_ids, atom_ids, pos_table, atom_table, ln_gamma, ln_beta):
    pid = position_ids.reshape(-1).astype(jnp.int32)
    aid = atom_ids.reshape(-1).astype(jnp.int32)
    x = input_embeds.reshape(_N, _D)
    gamma = ln_gamma.reshape(1, _D)
    beta = ln_beta.reshape(1, _D)

    bases = [sum(_CHUNKS[:j]) for j in range(len(_CHUNKS))]
    sums = [
        _sc_gather_chunk(cb, nt)(pos_table, atom_table, pid, aid)
        for cb, nt in zip(bases, _CHUNKS)
    ]
    buf = None
    for cb, nt, sum_j in zip(bases, _CHUNKS, sums):
        buf = _tc_chunk(cb, nt, buf, x, sum_j, gamma, beta)
    return buf.reshape(_B, _S, _D)


# speedup vs baseline: 1.0153x; 1.0153x over previous
import functools

import jax
import jax.numpy as jnp
from jax import lax
from jax.experimental import pallas as pl
from jax.experimental.pallas import tpu as pltpu
from jax.experimental.pallas import tpu_sc as plsc

_B, _S, _D = 4, 4096, 1024
_N = _B * _S
_NC, _NS = 2, 16
_NW = _NC * _NS
_CHUNKS = (4096, 4096, 4096, 4096)
_GC = 16
_LANES = 16
_ROWS = 1024
_EPS = 1e-12


def _sc_gather_chunk(cbase, ntok):
    mesh = plsc.VectorSubcoreMesh(core_axis_name="c", subcore_axis_name="s")
    _PER_W = ntok // _NW
    _NG = _PER_W // _GC

    @functools.partial(
        pl.kernel,
        mesh=mesh,
        out_type=jax.ShapeDtypeStruct((ntok, _D), jnp.float32),
        scratch_types=[
            pltpu.VMEM((_PER_W,), jnp.int32),
            pltpu.VMEM((_PER_W,), jnp.int32),
        ]
        + [pltpu.VMEM((_GC, _D), jnp.float32)] * 4
        + [pltpu.SemaphoreType.DMA] * 6,
    )
    def k(pos_tab, atom_tab, pid, aid, sum_out, pidx_v, aidx_v, *scr):
        bufp = scr[0:2]
        bufa = scr[2:4]
        gsemp = scr[4:6]
        gsema = scr[6:8]
        wsem = scr[8:10]
        wid = lax.axis_index("s") * _NC + lax.axis_index("c")
        base = wid * _PER_W
        pltpu.sync_copy(pid.at[pl.ds(cbase + base, _PER_W)], pidx_v)
        pltpu.sync_copy(aid.at[pl.ds(cbase + base, _PER_W)], aidx_v)

        def _gcopies(c):
            b = c % 2
            off = c * _GC
            return (
                pltpu.make_async_copy(
                    pos_tab.at[pidx_v.at[pl.ds(off, _GC)]], bufp[b], gsemp[b]
                ),
                pltpu.make_async_copy(
                    atom_tab.at[aidx_v.at[pl.ds(off, _GC)]], bufa[b], gsema[b]
                ),
            )

        def _wcopy(c):
            b = c % 2
            off = c * _GC
            return pltpu.make_async_copy(
                bufp[b], sum_out.at[pl.ds(base + off, _GC)], wsem[b]
            )

        def _add(c):
            b = c % 2
            bp, ba = bufp[b], bufa[b]

            @pl.loop(0, _GC)
            def _(r):
                for u in range(_D // _LANES):
                    sl = pl.ds(u * _LANES, _LANES)
                    bp[r, sl] = bp[r, sl] + ba[r, sl]

        def _process(c):
            for cp in _gcopies(c):
                cp.wait()
            _add(c)
            _wcopy(c).start()

        for c in range(_NG):
            if c >= 2:
                _wcopy(c - 2).wait()
            for cp in _gcopies(c):
                cp.start()
            if c >= 1:
                _process(c - 1)
        _process(_NG - 1)
        _wcopy(_NG - 2).wait()
        _wcopy(_NG - 1).wait()

    return k


def _addln_body(x_ref, s_ref, g_ref, b_ref, o_ref):
    v = x_ref[...] + s_ref[...]
    mean = jnp.mean(v, axis=-1, keepdims=True)
    vc = v - mean
    var = jnp.mean(vc * vc, axis=-1, keepdims=True)
    o_ref[...] = vc * lax.rsqrt(var + _EPS) * g_ref[...] + b_ref[...]


def _tc_chunk(rbase, ntok, buf, x, sum_j, gamma, beta):
    blk0 = rbase // _ROWS
    row_spec = pl.BlockSpec((_ROWS, _D), lambda i, blk0=blk0: (blk0 + i, 0))
    chunk_spec = pl.BlockSpec((_ROWS, _D), lambda i: (i, 0))
    vec_spec = pl.BlockSpec((1, _D), lambda i: (0, 0))
    common = dict(
        grid=(ntok // _ROWS,),
        out_specs=row_spec,
        out_shape=jax.ShapeDtypeStruct((_N, _D), jnp.float32),
    )
    if buf is None:
        return pl.pallas_call(
            _addln_body,
            in_specs=[row_spec, chunk_spec, vec_spec, vec_spec],
            **common,
        )(x, sum_j, gamma, beta)

    def body(buf_ref, x_ref, s_ref, g_ref, b_ref, o_ref):
        _addln_body(x_ref, s_ref, g_ref, b_ref, o_ref)

    return pl.pallas_call(
        body,
        in_specs=[
            pl.BlockSpec(memory_space=pl.ANY),
            row_spec,
            chunk_spec,
            vec_spec,
            vec_spec,
        ],
        input_output_aliases={0: 0},
        **common,
    )(buf, x, sum_j, gamma, beta)


def kernel(input_embeds, position_ids, atom_ids, pos_table, atom_table, ln_gamma, ln_beta):
    pid = position_ids.reshape(-1).astype(jnp.int32)
    aid = atom_ids.reshape(-1).astype(jnp.int32)
    x = input_embeds.reshape(_N, _D)
    gamma = ln_gamma.reshape(1, _D)
    beta = ln_beta.reshape(1, _D)

    bases = [sum(_CHUNKS[:j]) for j in range(len(_CHUNKS))]
    sums = [
        _sc_gather_chunk(cb, nt)(pos_table, atom_table, pid, aid)
        for cb, nt in zip(bases, _CHUNKS)
    ]
    buf = None
    for cb, nt, sum_j in zip(bases, _CHUNKS, sums):
        buf = _tc_chunk(cb, nt, buf, x, sum_j, gamma, beta)
    return buf.reshape(_B, _S, _D)


# speedup vs baseline: 1.0233x; 1.0079x over previous
import functools

import jax
import jax.numpy as jnp
from jax import lax
from jax.experimental import pallas as pl
from jax.experimental.pallas import tpu as pltpu
from jax.experimental.pallas import tpu_sc as plsc

_B, _S, _D = 4, 4096, 1024
_N = _B * _S
_NC, _NS = 2, 16
_NW = _NC * _NS
_CHUNKS = (4096, 4096, 4096, 4096)
_GC = 16
_LANES = 16
_ROWS = 1024
_EPS = 1e-12


def _sc_gather_chunk(cbase, ntok):
    mesh = plsc.VectorSubcoreMesh(core_axis_name="c", subcore_axis_name="s")
    _PER_W = ntok // _NW
    _NG = _PER_W // _GC

    @functools.partial(
        pl.kernel,
        mesh=mesh,
        out_type=jax.ShapeDtypeStruct((ntok, _D), jnp.float32),
        scratch_types=[
            pltpu.VMEM((_PER_W,), jnp.int32),
            pltpu.VMEM((_PER_W,), jnp.int32),
        ]
        + [pltpu.VMEM((_GC, _D), jnp.float32)] * 4
        + [pltpu.SemaphoreType.DMA] * 8,
    )
    def k(pos_tab, atom_tab, pid, aid, sum_out, pidx_v, aidx_v, *scr):
        bufp = scr[0:2]
        bufa = scr[2:4]
        gsemp = scr[4:6]
        gsema = scr[6:8]
        wsem = scr[8:10]
        isem = scr[10:12]
        wid = lax.axis_index("s") * _NC + lax.axis_index("c")
        base = wid * _PER_W
        ip = pltpu.make_async_copy(pid.at[pl.ds(cbase + base, _PER_W)], pidx_v, isem[0])
        ia = pltpu.make_async_copy(aid.at[pl.ds(cbase + base, _PER_W)], aidx_v, isem[1])
        ip.start()
        ia.start()
        ip.wait()
        ia.wait()

        def _gcopies(c):
            b = c % 2
            off = c * _GC
            return (
                pltpu.make_async_copy(
                    pos_tab.at[pidx_v.at[pl.ds(off, _GC)]], bufp[b], gsemp[b]
                ),
                pltpu.make_async_copy(
                    atom_tab.at[aidx_v.at[pl.ds(off, _GC)]], bufa[b], gsema[b]
                ),
            )

        def _wcopy(c):
            b = c % 2
            off = c * _GC
            return pltpu.make_async_copy(
                bufp[b], sum_out.at[pl.ds(base + off, _GC)], wsem[b]
            )

        def _add(c):
            b = c % 2
            bp, ba = bufp[b], bufa[b]

            @pl.loop(0, _GC)
            def _(r):
                for u in range(_D // _LANES):
                    sl = pl.ds(u * _LANES, _LANES)
                    bp[r, sl] = bp[r, sl] + ba[r, sl]

        def _process(c):
            for cp in _gcopies(c):
                cp.wait()
            _add(c)
            _wcopy(c).start()

        for c in range(_NG):
            if c >= 2:
                _wcopy(c - 2).wait()
            for cp in _gcopies(c):
                cp.start()
            if c >= 1:
                _process(c - 1)
        _process(_NG - 1)
        _wcopy(_NG - 2).wait()
        _wcopy(_NG - 1).wait()

    return k


def _addln_body(x_ref, s_ref, g_ref, b_ref, o_ref):
    v = x_ref[...] + s_ref[...]
    mean = jnp.mean(v, axis=-1, keepdims=True)
    vc = v - mean
    var = jnp.mean(vc * vc, axis=-1, keepdims=True)
    o_ref[...] = vc * lax.rsqrt(var + _EPS) * g_ref[...] + b_ref[...]


def _tc_chunk(rbase, ntok, buf, x, sum_j, gamma, beta):
    blk0 = rbase // _ROWS
    row_spec = pl.BlockSpec((_ROWS, _D), lambda i, blk0=blk0: (blk0 + i, 0))
    chunk_spec = pl.BlockSpec((_ROWS, _D), lambda i: (i, 0))
    vec_spec = pl.BlockSpec((1, _D), lambda i: (0, 0))
    common = dict(
        grid=(ntok // _ROWS,),
        out_specs=row_spec,
        out_shape=jax.ShapeDtypeStruct((_N, _D), jnp.float32),
    )
    if buf is None:
        return pl.pallas_call(
            _addln_body,
            in_specs=[row_spec, chunk_spec, vec_spec, vec_spec],
            **common,
        )(x, sum_j, gamma, beta)

    def body(buf_ref, x_ref, s_ref, g_ref, b_ref, o_ref):
        _addln_body(x_ref, s_ref, g_ref, b_ref, o_ref)

    return pl.pallas_call(
        body,
        in_specs=[
            pl.BlockSpec(memory_space=pl.ANY),
            row_spec,
            chunk_spec,
            vec_spec,
            vec_spec,
        ],
        input_output_aliases={0: 0},
        **common,
    )(buf, x, sum_j, gamma, beta)


def kernel(input_embeds, position_ids, atom_ids, pos_table, atom_table, ln_gamma, ln_beta):
    pid = position_ids.reshape(-1).astype(jnp.int32)
    aid = atom_ids.reshape(-1).astype(jnp.int32)
    x = input_embeds.reshape(_N, _D)
    gamma = ln_gamma.reshape(1, _D)
    beta = ln_beta.reshape(1, _D)

    bases = [sum(_CHUNKS[:j]) for j in range(len(_CHUNKS))]
    sums = [
        _sc_gather_chunk(cb, nt)(pos_table, atom_table, pid, aid)
        for cb, nt in zip(bases, _CHUNKS)
    ]
    buf = None
    for cb, nt, sum_j in zip(bases, _CHUNKS, sums):
        buf = _tc_chunk(cb, nt, buf, x, sum_j, gamma, beta)
    return buf.reshape(_B, _S, _D)
